# Initial kernel scaffold; baseline (speedup 1.0000x reference)
import functools

import jax
import jax.numpy as jnp
from jax import lax
from jax.experimental import pallas as pl
from jax.experimental.pallas import tpu as pltpu
from jax.experimental.pallas import tpu_sc as plsc

B, L, D = 4096, 50, 64
N = B * L
NW = 32
PER_W = N // NW
CH = 128
NCH = PER_W // CH


@jax.jit
def _poi_gather(idx2d, table):
    mesh = plsc.VectorSubcoreMesh(core_axis_name="c", subcore_axis_name="s")

    @functools.partial(
        pl.kernel,
        out_type=jax.ShapeDtypeStruct((N, D), jnp.float32),
        mesh=mesh,
        scratch_types=[
            pltpu.VMEM((NCH, CH), jnp.int32),
            pltpu.VMEM((2, CH, D), jnp.float32),
            pltpu.SemaphoreType.DMA,
            pltpu.SemaphoreType.DMA,
        ],
    )
    def k(idx_hbm, table_hbm, out_hbm, idx_v, rows_v, g_sem, o_sem):
        wid = lax.axis_index("s") * 2 + lax.axis_index("c")
        base = wid * PER_W
        pltpu.sync_copy(idx_hbm.at[pl.ds(wid * NCH, NCH)], idx_v)

        def gather(j, slot):
            return pltpu.async_copy(
                table_hbm.at[idx_v.at[j]], rows_v.at[slot], g_sem)

        def put(j, slot):
            return pltpu.async_copy(
                rows_v.at[slot], out_hbm.at[pl.ds(base + j * CH, CH)], o_sem)

        gather(0, 0).wait()

        def body(j, _):
            slot = j % 2
            gather(j + 1, 1 - slot).wait()
            put(j, slot).wait()
            return _

        lax.fori_loop(0, NCH - 1, body, 0)
        put(NCH - 1, (NCH - 1) % 2).wait()

    return k(idx2d, table)


def kernel(x, POI):
    idx2d = x.reshape(N).astype(jnp.int32).reshape(NW * NCH, CH)
    out = _poi_gather(idx2d, POI)
    return out.reshape(B, L, D)


# speedup vs baseline: 4.2714x; 4.2714x over previous
import functools

import jax
import jax.numpy as jnp
from jax import lax
from jax.experimental import pallas as pl
from jax.experimental.pallas import tpu as pltpu
from jax.experimental.pallas import tpu_sc as plsc

B, L, D = 4096, 50, 64
N = B * L
NW = 32
PER_W = N // NW
CH = 128
NCH = PER_W // CH


@jax.jit
def _poi_gather(idx2d, table):
    mesh = plsc.VectorSubcoreMesh(core_axis_name="c", subcore_axis_name="s")

    @functools.partial(
        pl.kernel,
        out_type=jax.ShapeDtypeStruct((N, D), jnp.float32),
        mesh=mesh,
        compiler_params=pltpu.CompilerParams(use_tc_tiling_on_sc=False),
        scratch_types=[
            pltpu.VMEM((NCH, CH), jnp.int32),
            pltpu.VMEM((2, CH, D), jnp.float32),
            pltpu.SemaphoreType.DMA,
            pltpu.SemaphoreType.DMA,
        ],
    )
    def k(idx_hbm, table_hbm, out_hbm, idx_v, rows_v, g_sem, o_sem):
        wid = lax.axis_index("s") * 2 + lax.axis_index("c")
        base = wid * PER_W
        pltpu.sync_copy(idx_hbm.at[wid], idx_v)

        def gather(j, slot):
            return pltpu.async_copy(
                table_hbm.at[idx_v.at[j]], rows_v.at[slot], g_sem)

        def put(j, slot):
            return pltpu.async_copy(
                rows_v.at[slot], out_hbm.at[pl.ds(base + j * CH, CH)], o_sem)

        gather(0, 0).wait()

        def body(j, _):
            slot = j % 2
            g = gather(j + 1, 1 - slot)
            p = put(j, slot)
            p.wait()
            g.wait()
            return _

        lax.fori_loop(0, NCH - 1, body, 0)
        put(NCH - 1, (NCH - 1) % 2).wait()

    return k(idx2d, table)


def kernel(x, POI):
    idx3d = x.reshape(N).astype(jnp.int32).reshape(NW, NCH, CH)
    out = _poi_gather(idx3d, POI)
    return out.reshape(B, L, D)


# speedup vs baseline: 4.6647x; 1.0921x over previous
import functools

import jax
import jax.numpy as jnp
from jax import lax
from jax.experimental import pallas as pl
from jax.experimental.pallas import tpu as pltpu
from jax.experimental.pallas import tpu_sc as plsc

B, L, D = 4096, 50, 64
N = B * L
NW = 32
PER_W = N // NW
CH = 128
NCH = PER_W // CH
K = 5
NPH = NCH // K


@jax.jit
def _poi_gather(idx3d, table):
    mesh = plsc.VectorSubcoreMesh(core_axis_name="c", subcore_axis_name="s")

    @functools.partial(
        pl.kernel,
        out_type=jax.ShapeDtypeStruct((N, D), jnp.float32),
        mesh=mesh,
        compiler_params=pltpu.CompilerParams(use_tc_tiling_on_sc=False),
        scratch_types=[
            pltpu.VMEM((NCH, CH), jnp.int32),
            pltpu.VMEM((2, K, CH, D), jnp.float32),
            pltpu.SemaphoreType.DMA,
            pltpu.SemaphoreType.DMA,
        ],
    )
    def k(idx_hbm, table_hbm, out_hbm, idx_v, rows_v, g_sem, o_sem):
        wid = lax.axis_index("s") * 2 + lax.axis_index("c")
        base = wid * PER_W
        pltpu.sync_copy(idx_hbm.at[wid], idx_v)

        def fire(p, bank):
            for b in range(K):
                pltpu.async_copy(
                    table_hbm.at[idx_v.at[p * K + b]], rows_v.at[bank, b],
                    g_sem)

        def wait_gathers():
            for _ in range(K):
                pltpu.make_async_copy(
                    table_hbm.at[idx_v.at[0]], rows_v.at[0, 0], g_sem).wait()

        def puts(p, bank):
            for b in range(K):
                pltpu.async_copy(
                    rows_v.at[bank, b],
                    out_hbm.at[pl.ds(base + (p * K + b) * CH, CH)], o_sem)

        def wait_puts():
            for _ in range(K):
                pltpu.make_async_copy(
                    rows_v.at[0, 0], out_hbm.at[pl.ds(base, CH)],
                    o_sem).wait()

        fire(0, 0)

        def body(i, carry):
            for q in range(2):
                p = 2 * i + q

                @pl.when(p > 0)
                def _drain():
                    wait_puts()

                @pl.when(p < NPH - 1)
                def _prefetch():
                    fire(p + 1, 1 - q)

                wait_gathers()
                puts(p, q)
            return carry

        lax.fori_loop(0, NPH // 2, body, 0)
        wait_puts()

    return k(idx3d, table)


def kernel(x, POI):
    idx3d = x.reshape(N).astype(jnp.int32).reshape(NW, NCH, CH)
    out = _poi_gather(idx3d, POI)
    return out.reshape(B, L, D)


# speedup vs baseline: 4.6668x; 1.0004x over previous
import functools

import jax
import jax.numpy as jnp
from jax import lax
from jax.experimental import pallas as pl
from jax.experimental.pallas import tpu as pltpu
from jax.experimental.pallas import tpu_sc as plsc

B, L, D = 4096, 50, 64
NW = 32
RPW = B // NW
K = 8
NPH = RPW // K


@jax.jit
def _poi_gather(x, table):
    mesh = plsc.VectorSubcoreMesh(core_axis_name="c", subcore_axis_name="s")

    @functools.partial(
        pl.kernel,
        out_type=jax.ShapeDtypeStruct((B, L, D), jnp.float32),
        mesh=mesh,
        compiler_params=pltpu.CompilerParams(use_tc_tiling_on_sc=False),
        scratch_types=[
            pltpu.VMEM((RPW, L), jnp.int32),
            pltpu.VMEM((2, K, L, D), jnp.float32),
            pltpu.SemaphoreType.DMA,
            pltpu.SemaphoreType.DMA,
        ],
    )
    def k(x_hbm, table_hbm, out_hbm, idx_v, rows_v, g_sem, o_sem):
        wid = lax.axis_index("s") * 2 + lax.axis_index("c")
        base = wid * RPW
        pltpu.sync_copy(x_hbm.at[pl.ds(base, RPW)], idx_v)

        def fire(p, bank):
            for b in range(K):
                pltpu.async_copy(
                    table_hbm.at[idx_v.at[p * K + b]], rows_v.at[bank, b],
                    g_sem)

        def wait_gathers():
            for _ in range(K):
                pltpu.make_async_copy(
                    table_hbm.at[idx_v.at[0]], rows_v.at[0, 0], g_sem).wait()

        def puts(p, bank):
            for b in range(K):
                pltpu.async_copy(
                    rows_v.at[bank, b], out_hbm.at[base + p * K + b], o_sem)

        def wait_puts():
            for _ in range(K):
                pltpu.make_async_copy(
                    rows_v.at[0, 0], out_hbm.at[0], o_sem).wait()

        fire(0, 0)

        def body(i, carry):
            for q in range(2):
                p = 2 * i + q

                @pl.when(p > 0)
                def _drain():
                    wait_puts()

                @pl.when(p < NPH - 1)
                def _prefetch():
                    fire(p + 1, 1 - q)

                wait_gathers()
                puts(p, q)
            return carry

        lax.fori_loop(0, NPH // 2, body, 0)
        wait_puts()

    return k(x, table)


def kernel(x, POI):
    return _poi_gather(x.astype(jnp.int32), POI)


# speedup vs baseline: 7.0144x; 1.5030x over previous
import functools

import jax
import jax.numpy as jnp
from jax import lax
from jax.experimental import pallas as pl
from jax.experimental.pallas import tpu as pltpu
from jax.experimental.pallas import tpu_sc as plsc

B, L, D = 4096, 50, 64
NW = 32
RPW = B // NW
K = 8
NPH = RPW // K


@jax.jit
def _poi_gather(x, table):
    mesh = plsc.VectorSubcoreMesh(core_axis_name="c", subcore_axis_name="s")

    @functools.partial(
        pl.kernel,
        out_type=jax.ShapeDtypeStruct((B, 56, 128), jnp.float32),
        mesh=mesh,
        compiler_params=pltpu.CompilerParams(use_tc_tiling_on_sc=False),
        scratch_types=[
            pltpu.VMEM((RPW, L), jnp.int32),
            pltpu.VMEM((2, K, L, D), jnp.float32),
            pltpu.SemaphoreType.DMA,
            pltpu.SemaphoreType.DMA,
        ],
    )
    def k(x_hbm, table_hbm, out_hbm, idx_v, rows_v, g_sem, o_sem):
        wid = lax.axis_index("s") * 2 + lax.axis_index("c")
        base = wid * RPW
        pltpu.sync_copy(x_hbm.at[pl.ds(base, RPW)], idx_v)

        def fire(p, bank):
            for b in range(K):
                pltpu.async_copy(
                    table_hbm.at[idx_v.at[p * K + b]], rows_v.at[bank, b],
                    g_sem)

        def wait_gathers():
            for _ in range(K):
                pltpu.make_async_copy(
                    table_hbm.at[idx_v.at[0]], rows_v.at[0, 0], g_sem).wait()

        def puts(p, bank):
            for b in range(K):
                pltpu.async_copy(
                    rows_v.at[bank, b],
                    out_hbm.at[base + p * K + b, pl.ds(0, L), pl.ds(0, D)],
                    o_sem)

        def wait_puts():
            for _ in range(K):
                pltpu.make_async_copy(
                    rows_v.at[0, 0],
                    out_hbm.at[0, pl.ds(0, L), pl.ds(0, D)], o_sem).wait()

        fire(0, 0)

        def body(i, carry):
            for q in range(2):
                p = 2 * i + q

                @pl.when(p > 0)
                def _drain():
                    wait_puts()

                @pl.when(p < NPH - 1)
                def _prefetch():
                    fire(p + 1, 1 - q)

                wait_gathers()
                puts(p, q)
            return carry

        lax.fori_loop(0, NPH // 2, body, 0)
        wait_puts()

    return k(x, table)


def kernel(x, POI):
    big = _poi_gather(x.astype(jnp.int32), POI)
    return big[:, :L, :D]
